# baseline (speedup 1.0000x reference)
import functools

import jax
import jax.numpy as jnp
from jax import lax
from jax.experimental import pallas as pl
from jax.experimental.pallas import tpu as pltpu
from jax.experimental.pallas import tpu_sc as plsc

DIM = 32
L = 20
K = 5
M = 5
B = 4096
NB = 16
BOUND = 250000
NP = (L - K) * K

NW = 32
B_PER = B // NW
GB = 16
NG = B_PER // GB
WROWS = GB * L
NROWS = GB * (L - K) * M


def _sc_body(table, widx, nidx, out_p, out_n,
             widx_v, nidx_v, wbuf, nbuf, dbuf_p, dbuf_n, sem):
    wid = lax.axis_index("s") * 2 + lax.axis_index("c")
    lane = lax.iota(jnp.int32, 16)
    zeros16 = jnp.zeros((16,), jnp.float32)

    for r in range(GB):
        for cblk in range(NP // 16, 8):
            dbuf_p[r, pl.ds(cblk * 16, 16)] = zeros16
            dbuf_n[r, pl.ds(cblk * 16, 16)] = zeros16

    def group(g, _):
        b0 = wid * B_PER + g * GB
        pltpu.sync_copy(widx.at[pl.ds(b0 * L, WROWS)], widx_v)
        pltpu.async_copy(table.at[widx_v], wbuf, sem).wait()
        pltpu.sync_copy(nidx.at[pl.ds(b0 * NP, NROWS)], nidx_v)
        pltpu.async_copy(table.at[nidx_v], nbuf, sem).wait()

        wrow_base = lane * L
        nrow_base = lane * NP

        def per_i(i, _):
            w_rows = wrow_base + i
            acc_p = [zeros16] * K
            acc_n = [zeros16] * M
            for d in range(DIM):
                dvec = jnp.full((16,), d, jnp.int32)
                wv = plsc.load_gather(wbuf, [w_rows, dvec])
                cps = [plsc.load_gather(wbuf, [wrow_base + (i + 1 + k), dvec])
                       for k in range(K)]
                cns = [plsc.load_gather(nbuf, [nrow_base + (i * M + m), dvec])
                       for m in range(M)]
                for k in range(K):
                    acc_p[k] = acc_p[k] + wv * cps[k]
                for m in range(M):
                    acc_n[m] = acc_n[m] + wv * cns[m]
            for k in range(K):
                col = jnp.full((16,), i * K + k, jnp.int32)
                plsc.store_scatter(dbuf_p, [lane, col], acc_p[k])
            for m in range(M):
                col = jnp.full((16,), i * M + m, jnp.int32)
                plsc.store_scatter(dbuf_n, [lane, col], acc_n[m])
            return 0

        lax.fori_loop(0, L - K, per_i, 0)
        pltpu.sync_copy(dbuf_p, out_p.at[pl.ds(b0, GB)])
        pltpu.sync_copy(dbuf_n, out_n.at[pl.ds(b0, GB)])
        return 0

    lax.fori_loop(0, NG, group, 0)


@functools.cache
def _sc_dots():
    return pl.kernel(
        _sc_body,
        mesh=plsc.VectorSubcoreMesh(core_axis_name="c", subcore_axis_name="s"),
        out_type=[
            jax.ShapeDtypeStruct((B, 128), jnp.float32),
            jax.ShapeDtypeStruct((B, 128), jnp.float32),
        ],
        scratch_types=[
            pltpu.VMEM((WROWS,), jnp.int32),
            pltpu.VMEM((NROWS,), jnp.int32),
            pltpu.VMEM((WROWS, DIM), jnp.float32),
            pltpu.VMEM((NROWS, DIM), jnp.float32),
            pltpu.VMEM((GB, 128), jnp.float32),
            pltpu.VMEM((GB, 128), jnp.float32),
            pltpu.SemaphoreType.DMA,
        ],
        compiler_params=pltpu.CompilerParams(
            use_tc_tiling_on_sc=False, needs_layout_passes=False),
    )


def _type_of(t):
    return ((t >= BOUND).astype(jnp.int32)
            + (t >= 2 * BOUND).astype(jnp.int32)
            + (t >= 3 * BOUND).astype(jnp.int32))


def _softplus(x):
    return jnp.maximum(x, 0.0) + jnp.log(1.0 + jnp.exp(-jnp.abs(x)))


def _tc_body(cen_ref, pos_ref, neg_ref, pd_ref, nd_ref, out_ref):
    pi = pl.program_id(0)

    @pl.when(pi == 0)
    def _():
        out_ref[...] = jnp.zeros_like(out_ref)

    pos_dots = pd_ref[...][:, :NP]
    neg_dots = nd_ref[...][:, :NP]

    loss_all = jnp.concatenate(
        [_softplus(-pos_dots), _softplus(neg_dots)], axis=1)

    ct = _type_of(cen_ref[...])
    pt = _type_of(pos_ref[...])
    nt = _type_of(neg_ref[...])
    bins_all = jnp.concatenate([4 * ct + pt, 4 * ct + nt], axis=1)

    lane = lax.broadcasted_iota(jnp.int32, (1, NB), 1)
    srow = jnp.zeros((1, NB), jnp.float32)
    crow = jnp.zeros((1, NB), jnp.float32)
    for t in range(NB):
        mask = bins_all == t
        s_t = jnp.sum(jnp.where(mask, loss_all, 0.0))
        c_t = jnp.sum(mask.astype(jnp.float32))
        sel = lane == t
        srow += jnp.where(sel, s_t, 0.0)
        crow += jnp.where(sel, c_t, 0.0)

    out_ref[...] += jnp.concatenate([srow, crow], axis=0)


def kernel(walk, negative, node_embedding):
    walk_flat = walk.reshape(-1)
    neg_flat = negative.reshape(-1)
    pos_dots, neg_dots = _sc_dots()(node_embedding, walk_flat, neg_flat)

    cen_ids = jnp.repeat(walk[:, :L - K], K, axis=1)
    pos_ids = jnp.concatenate(
        [walk[:, i + 1:i + K + 1] for i in range(L - K)], axis=1)
    neg_ids = negative.reshape(B, NP)

    BB = 1024
    grid = B // BB
    out = pl.pallas_call(
        _tc_body,
        grid=(grid,),
        in_specs=[
            pl.BlockSpec((BB, NP), lambda i: (i, 0)),
            pl.BlockSpec((BB, NP), lambda i: (i, 0)),
            pl.BlockSpec((BB, NP), lambda i: (i, 0)),
            pl.BlockSpec((BB, 128), lambda i: (i, 0)),
            pl.BlockSpec((BB, 128), lambda i: (i, 0)),
        ],
        out_specs=pl.BlockSpec((2, NB), lambda i: (0, 0)),
        out_shape=jax.ShapeDtypeStruct((2, NB), jnp.float32),
    )(cen_ids, pos_ids, neg_ids, pos_dots, neg_dots)

    sums = out[0]
    cnts = out[1]
    total = jnp.float32(2 * B * (L - K) * K)
    loss = jnp.sum(sums) / total
    return loss, sums / cnts


# speedup vs baseline: 1.4241x; 1.4241x over previous
import functools

import jax
import jax.numpy as jnp
from jax import lax
from jax.experimental import pallas as pl
from jax.experimental.pallas import tpu as pltpu
from jax.experimental.pallas import tpu_sc as plsc

DIM = 32
L = 20
K = 5
M = 5
B = 4096
NB = 16
BOUND = 250000
NP = (L - K) * K

NW = 32
B_PER = B // NW
GB = 16
NG = B_PER // GB
WROWS = GB * L
NROWS = GB * (L - K) * M


def _sc_body(table, widx, nidx, out_p, out_n,
             widx_v, nidx_v, wbuf, nbuf, dbuf_p, dbuf_n, sem):
    wid = lax.axis_index("s") * 2 + lax.axis_index("c")
    lane = lax.iota(jnp.int32, 16)
    zeros16 = jnp.zeros((16,), jnp.float32)

    for r in range(GB):
        for cblk in range(NP // 16, 8):
            dbuf_p[r, pl.ds(cblk * 16, 16)] = zeros16
            dbuf_n[r, pl.ds(cblk * 16, 16)] = zeros16

    def group(g, _):
        b0 = wid * B_PER + g * GB
        pltpu.sync_copy(widx.at[pl.ds(b0 * L, WROWS)], widx_v)
        pltpu.async_copy(table.at[widx_v], wbuf, sem).wait()
        pltpu.sync_copy(nidx.at[pl.ds(b0 * NP, NROWS)], nidx_v)
        pltpu.async_copy(table.at[nidx_v], nbuf, sem).wait()

        wrow_base = lane * L
        nrow_base = lane * NP

        def per_i(i, _):
            w_rows = wrow_base + i
            acc_p = [zeros16] * K
            acc_n = [zeros16] * M
            for d in range(DIM):
                dvec = (lane + d) & (DIM - 1)
                wv = plsc.load_gather(wbuf, [w_rows, dvec])
                cps = [plsc.load_gather(wbuf, [wrow_base + (i + 1 + k), dvec])
                       for k in range(K)]
                cns = [plsc.load_gather(nbuf, [nrow_base + (i * M + m), dvec])
                       for m in range(M)]
                for k in range(K):
                    acc_p[k] = acc_p[k] + wv * cps[k]
                for m in range(M):
                    acc_n[m] = acc_n[m] + wv * cns[m]
            for k in range(K):
                col = jnp.full((16,), i * K + k, jnp.int32)
                plsc.store_scatter(dbuf_p, [lane, col], acc_p[k])
            for m in range(M):
                col = jnp.full((16,), i * M + m, jnp.int32)
                plsc.store_scatter(dbuf_n, [lane, col], acc_n[m])
            return 0

        lax.fori_loop(0, L - K, per_i, 0)
        pltpu.sync_copy(dbuf_p, out_p.at[pl.ds(b0, GB)])
        pltpu.sync_copy(dbuf_n, out_n.at[pl.ds(b0, GB)])
        return 0

    lax.fori_loop(0, NG, group, 0)


@functools.cache
def _sc_dots():
    return pl.kernel(
        _sc_body,
        mesh=plsc.VectorSubcoreMesh(core_axis_name="c", subcore_axis_name="s"),
        out_type=[
            jax.ShapeDtypeStruct((B, 128), jnp.float32),
            jax.ShapeDtypeStruct((B, 128), jnp.float32),
        ],
        scratch_types=[
            pltpu.VMEM((WROWS,), jnp.int32),
            pltpu.VMEM((NROWS,), jnp.int32),
            pltpu.VMEM((WROWS, DIM), jnp.float32),
            pltpu.VMEM((NROWS, DIM), jnp.float32),
            pltpu.VMEM((GB, 128), jnp.float32),
            pltpu.VMEM((GB, 128), jnp.float32),
            pltpu.SemaphoreType.DMA,
        ],
        compiler_params=pltpu.CompilerParams(
            use_tc_tiling_on_sc=False, needs_layout_passes=False),
    )


def _type_of(t):
    return ((t >= BOUND).astype(jnp.int32)
            + (t >= 2 * BOUND).astype(jnp.int32)
            + (t >= 3 * BOUND).astype(jnp.int32))


def _softplus(x):
    return jnp.maximum(x, 0.0) + jnp.log(1.0 + jnp.exp(-jnp.abs(x)))


def _tc_body(cen_ref, pos_ref, neg_ref, pd_ref, nd_ref, out_ref):
    pi = pl.program_id(0)

    @pl.when(pi == 0)
    def _():
        out_ref[...] = jnp.zeros_like(out_ref)

    pos_dots = pd_ref[...][:, :NP]
    neg_dots = nd_ref[...][:, :NP]

    loss_all = jnp.concatenate(
        [_softplus(-pos_dots), _softplus(neg_dots)], axis=1)

    ct = _type_of(cen_ref[...])
    pt = _type_of(pos_ref[...])
    nt = _type_of(neg_ref[...])
    bins_all = jnp.concatenate([4 * ct + pt, 4 * ct + nt], axis=1)

    lane = lax.broadcasted_iota(jnp.int32, (1, NB), 1)
    srow = jnp.zeros((1, NB), jnp.float32)
    crow = jnp.zeros((1, NB), jnp.float32)
    for t in range(NB):
        mask = bins_all == t
        s_t = jnp.sum(jnp.where(mask, loss_all, 0.0))
        c_t = jnp.sum(mask.astype(jnp.float32))
        sel = lane == t
        srow += jnp.where(sel, s_t, 0.0)
        crow += jnp.where(sel, c_t, 0.0)

    out_ref[...] += jnp.concatenate([srow, crow], axis=0)


def kernel(walk, negative, node_embedding):
    walk_flat = walk.reshape(-1)
    neg_flat = negative.reshape(-1)
    pos_dots, neg_dots = _sc_dots()(node_embedding, walk_flat, neg_flat)

    cen_ids = jnp.repeat(walk[:, :L - K], K, axis=1)
    pos_ids = jnp.concatenate(
        [walk[:, i + 1:i + K + 1] for i in range(L - K)], axis=1)
    neg_ids = negative.reshape(B, NP)

    BB = 1024
    grid = B // BB
    out = pl.pallas_call(
        _tc_body,
        grid=(grid,),
        in_specs=[
            pl.BlockSpec((BB, NP), lambda i: (i, 0)),
            pl.BlockSpec((BB, NP), lambda i: (i, 0)),
            pl.BlockSpec((BB, NP), lambda i: (i, 0)),
            pl.BlockSpec((BB, 128), lambda i: (i, 0)),
            pl.BlockSpec((BB, 128), lambda i: (i, 0)),
        ],
        out_specs=pl.BlockSpec((2, NB), lambda i: (0, 0)),
        out_shape=jax.ShapeDtypeStruct((2, NB), jnp.float32),
    )(cen_ids, pos_ids, neg_ids, pos_dots, neg_dots)

    sums = out[0]
    cnts = out[1]
    total = jnp.float32(2 * B * (L - K) * K)
    loss = jnp.sum(sums) / total
    return loss, sums / cnts


# speedup vs baseline: 1.5139x; 1.0630x over previous
import functools

import jax
import jax.numpy as jnp
from jax import lax
from jax.experimental import pallas as pl
from jax.experimental.pallas import tpu as pltpu
from jax.experimental.pallas import tpu_sc as plsc

DIM = 32
L = 20
K = 5
M = 5
B = 4096
NB = 16
BOUND = 250000
NP = (L - K) * K

NW = 32
B_PER = B // NW
GB = 16
NG = B_PER // GB
WROWS = GB * L
NROWS = GB * (L - K) * M
DCOL = 129


def _sc_body(table, widx, nidx, out_p, out_n,
             widx_v, nidx_v, wbuf, nbuf, dbuf_p, dbuf_n, sems):
    wid = lax.axis_index("s") * 2 + lax.axis_index("c")
    lane = lax.iota(jnp.int32, 16)
    zeros16 = jnp.zeros((16,), jnp.float32)

    def start(g, p):
        b0 = wid * B_PER + g * GB
        pltpu.sync_copy(widx.at[pl.ds(b0 * L, WROWS)], widx_v.at[p])
        pltpu.async_copy(table.at[widx_v.at[p]], wbuf.at[p], sems.at[p])
        pltpu.sync_copy(nidx.at[pl.ds(b0 * NP, NROWS)], nidx_v.at[p])
        pltpu.async_copy(table.at[nidx_v.at[p]], nbuf.at[p], sems.at[p])

    def drain(p):
        pltpu.make_async_copy(table.at[widx_v.at[p]], wbuf.at[p],
                              sems.at[p]).wait()
        pltpu.make_async_copy(table.at[nidx_v.at[p]], nbuf.at[p],
                              sems.at[p]).wait()

    def process(g, p):
        b0 = wid * B_PER + g * GB
        wrow_base = lane * L
        nrow_base = lane * NP
        wb = wbuf.at[p]
        nb = nbuf.at[p]

        def per_i(i, _):
            w_rows = wrow_base + i
            acc_p = [zeros16] * K
            acc_n = [zeros16] * M
            for d in range(DIM):
                dvec = (lane + d) & (DIM - 1)
                wv = plsc.load_gather(wb, [w_rows, dvec])
                cps = [plsc.load_gather(wb, [wrow_base + (i + 1 + k), dvec])
                       for k in range(K)]
                cns = [plsc.load_gather(nb, [nrow_base + (i * M + m), dvec])
                       for m in range(M)]
                for k in range(K):
                    acc_p[k] = acc_p[k] + wv * cps[k]
                for m in range(M):
                    acc_n[m] = acc_n[m] + wv * cns[m]
            for k in range(K):
                col = jnp.full((16,), i * K + k, jnp.int32)
                plsc.store_scatter(dbuf_p, [lane, col], acc_p[k])
            for m in range(M):
                col = jnp.full((16,), i * M + m, jnp.int32)
                plsc.store_scatter(dbuf_n, [lane, col], acc_n[m])
            return 0

        lax.fori_loop(0, L - K, per_i, 0)
        pltpu.sync_copy(dbuf_p.at[:, pl.ds(0, 128)], out_p.at[pl.ds(b0, GB)])
        pltpu.sync_copy(dbuf_n.at[:, pl.ds(0, 128)], out_n.at[pl.ds(b0, GB)])

    for r in range(GB):
        for cblk in range(NP // 16, 8):
            dbuf_p[r, pl.ds(cblk * 16, 16)] = zeros16
            dbuf_n[r, pl.ds(cblk * 16, 16)] = zeros16

    start(0, 0)

    def two_groups(g0, _):
        for p in range(2):
            g = g0 + p

            @pl.when(g + 1 < NG)
            def _():
                start(g + 1, 1 - p)

            drain(p)
            process(g, p)
        return 0

    def loop_body(step, _):
        two_groups(step * 2, None)
        return 0

    lax.fori_loop(0, NG // 2, loop_body, 0)


@functools.cache
def _sc_dots():
    return pl.kernel(
        _sc_body,
        mesh=plsc.VectorSubcoreMesh(core_axis_name="c", subcore_axis_name="s"),
        out_type=[
            jax.ShapeDtypeStruct((B, 128), jnp.float32),
            jax.ShapeDtypeStruct((B, 128), jnp.float32),
        ],
        scratch_types=[
            pltpu.VMEM((2, WROWS), jnp.int32),
            pltpu.VMEM((2, NROWS), jnp.int32),
            pltpu.VMEM((2, WROWS, DIM), jnp.float32),
            pltpu.VMEM((2, NROWS, DIM), jnp.float32),
            pltpu.VMEM((GB, DCOL), jnp.float32),
            pltpu.VMEM((GB, DCOL), jnp.float32),
            pltpu.SemaphoreType.DMA((2,)),
        ],
        compiler_params=pltpu.CompilerParams(
            use_tc_tiling_on_sc=False, needs_layout_passes=False),
    )


def _type_of(t):
    return ((t >= BOUND).astype(jnp.int32)
            + (t >= 2 * BOUND).astype(jnp.int32)
            + (t >= 3 * BOUND).astype(jnp.int32))


def _softplus(x):
    return jnp.maximum(x, 0.0) + jnp.log(1.0 + jnp.exp(-jnp.abs(x)))


def _tc_body(walk_ref, negid_ref, pd_ref, nd_ref, out_ref):
    pi = pl.program_id(0)

    @pl.when(pi == 0)
    def _():
        out_ref[...] = jnp.zeros_like(out_ref)

    pos_dots = pd_ref[...][:, :NP]
    neg_dots = nd_ref[...][:, :NP]

    loss_all = jnp.concatenate(
        [_softplus(-pos_dots), _softplus(neg_dots)], axis=1)

    wt = _type_of(walk_ref[...]).astype(jnp.float32)
    nt = _type_of(negid_ref[...]).astype(jnp.float32)

    ri = lax.broadcasted_iota(jnp.int32, (L - K, NP), 0)
    ci = lax.broadcasted_iota(jnp.int32, (L - K, NP), 1)
    rep = (ri == ci // K).astype(jnp.float32)
    rj = lax.broadcasted_iota(jnp.int32, (L, NP), 0)
    cj = lax.broadcasted_iota(jnp.int32, (L, NP), 1)
    shift = (rj == cj // K + 1 + cj % K).astype(jnp.float32)

    ct = jnp.dot(wt[:, :L - K], rep, preferred_element_type=jnp.float32)
    ptv = jnp.dot(wt, shift, preferred_element_type=jnp.float32)

    bins_all = jnp.concatenate([4.0 * ct + ptv, 4.0 * ct + nt], axis=1)

    lane = lax.broadcasted_iota(jnp.int32, (1, NB), 1)
    srow = jnp.zeros((1, NB), jnp.float32)
    crow = jnp.zeros((1, NB), jnp.float32)
    for t in range(NB):
        mask = bins_all == float(t)
        s_t = jnp.sum(jnp.where(mask, loss_all, 0.0))
        c_t = jnp.sum(mask.astype(jnp.float32))
        sel = lane == t
        srow += jnp.where(sel, s_t, 0.0)
        crow += jnp.where(sel, c_t, 0.0)

    out_ref[...] += jnp.concatenate([srow, crow], axis=0)


def kernel(walk, negative, node_embedding):
    walk_flat = walk.reshape(-1)
    neg_flat = negative.reshape(-1)
    pos_dots, neg_dots = _sc_dots()(node_embedding, walk_flat, neg_flat)

    neg_ids = negative.reshape(B, NP)

    BB = 1024
    grid = B // BB
    out = pl.pallas_call(
        _tc_body,
        grid=(grid,),
        in_specs=[
            pl.BlockSpec((BB, L), lambda i: (i, 0)),
            pl.BlockSpec((BB, NP), lambda i: (i, 0)),
            pl.BlockSpec((BB, 128), lambda i: (i, 0)),
            pl.BlockSpec((BB, 128), lambda i: (i, 0)),
        ],
        out_specs=pl.BlockSpec((2, NB), lambda i: (0, 0)),
        out_shape=jax.ShapeDtypeStruct((2, NB), jnp.float32),
    )(walk, neg_ids, pos_dots, neg_dots)

    sums = out[0]
    cnts = out[1]
    total = jnp.float32(2 * B * (L - K) * K)
    loss = jnp.sum(sums) / total
    return loss, sums / cnts
